# baseline (speedup 1.0000x reference)
import functools

import jax
import jax.numpy as jnp
from jax import lax
from jax.experimental import pallas as pl
from jax.experimental.pallas import tpu as pltpu
from jax.experimental.pallas import tpu_sc as plsc

SEQ_LEN = 200
DIM = 64
NC = 2
NS = 16
NW = NC * NS
HALF = SEQ_LEN // 2


def _body(idx_hbm, table_hbm, pos_hbm, out_hbm, idx_v, rows_v, pos_v, sem):
    wid = lax.axis_index("s") * NC + lax.axis_index("c")
    batch = idx_hbm.shape[0]
    seq_per_w = batch // NW

    pltpu.sync_copy(pos_hbm, pos_v)

    def one_seq(i, carry):
        seq = wid * seq_per_w + i
        pltpu.sync_copy(idx_hbm.at[seq], idx_v)
        cp0 = pltpu.async_copy(table_hbm.at[idx_v.at[0]],
                               rows_v.at[pl.ds(0, HALF)], sem)
        cp1 = pltpu.async_copy(table_hbm.at[idx_v.at[1]],
                               rows_v.at[pl.ds(HALF, HALF)], sem)
        cp0.wait()
        cp1.wait()

        def add_row(r, c):
            for d in range(DIM // 16):
                sl = pl.ds(d * 16, 16)
                rows_v[r, sl] = rows_v[r, sl] + pos_v[r, sl]
            return c

        lax.fori_loop(0, SEQ_LEN, add_row, 0, unroll=4)
        pltpu.sync_copy(rows_v, out_hbm.at[seq])
        return carry

    lax.fori_loop(0, seq_per_w, one_seq, 0)


def kernel(inputs, token_table, pos_table):
    batch, seq_len = inputs.shape
    assert seq_len == SEQ_LEN and batch % NW == 0
    idx3 = inputs.reshape(batch, 2, HALF).astype(jnp.int32)

    mesh = plsc.VectorSubcoreMesh(
        core_axis_name="c", subcore_axis_name="s",
        num_cores=NC, num_subcores=NS)

    run = pl.kernel(
        _body,
        out_type=jax.ShapeDtypeStruct((batch, SEQ_LEN, DIM), jnp.float32),
        mesh=mesh,
        scratch_types=[
            pltpu.VMEM((2, HALF), jnp.int32),
            pltpu.VMEM((SEQ_LEN, DIM), jnp.float32),
            pltpu.VMEM((SEQ_LEN, DIM), jnp.float32),
            pltpu.SemaphoreType.DMA,
        ],
        compiler_params=pltpu.CompilerParams(use_tc_tiling_on_sc=False),
    )
    return run(idx3, token_table, pos_table)


# speedup vs baseline: 1.1815x; 1.1815x over previous
import jax
import jax.numpy as jnp
from jax import lax
from jax.experimental import pallas as pl
from jax.experimental.pallas import tpu as pltpu
from jax.experimental.pallas import tpu_sc as plsc

SEQ_LEN = 200
DIM = 64
NC = 2
NS = 16
NW = NC * NS
HALF = SEQ_LEN // 2
NB = 4
AHEAD = 2


def _body(idx_hbm, table_hbm, pos_hbm, out_hbm,
          idx_v, pos_v, rows, gsems, ssems):
    wid = lax.axis_index("s") * NC + lax.axis_index("c")
    batch = idx_hbm.shape[0]
    spw = batch // NW
    base = wid * spw

    pltpu.sync_copy(idx_hbm.at[pl.ds(base, spw)], idx_v)
    pltpu.sync_copy(pos_hbm, pos_v)

    def fire(s, b):
        pltpu.async_copy(table_hbm.at[idx_v.at[s, 0]],
                         rows[b].at[pl.ds(0, HALF)], gsems[b])
        pltpu.async_copy(table_hbm.at[idx_v.at[s, 1]],
                         rows[b].at[pl.ds(HALF, HALF)], gsems[b])

    def gather_wait(s, b):
        pltpu.make_async_copy(table_hbm.at[idx_v.at[s, 0]],
                              rows[b].at[pl.ds(0, HALF)], gsems[b]).wait()
        pltpu.make_async_copy(table_hbm.at[idx_v.at[s, 1]],
                              rows[b].at[pl.ds(HALF, HALF)], gsems[b]).wait()

    def store_wait(s, b):
        pltpu.make_async_copy(rows[b], out_hbm.at[base + s], ssems[b]).wait()

    for b in range(AHEAD):
        fire(b, b)

    def one_group(g, carry):
        s0 = g * NB
        for b in range(NB):
            s = s0 + b
            sf = s + AHEAD
            bf = (b + AHEAD) % NB
            @pl.when(jnp.logical_and(sf < spw, sf >= NB))
            def _():
                store_wait(sf - NB, bf)

            @pl.when(sf < spw)
            def _():
                fire(sf, bf)

            gather_wait(s, b)

            def add_row(r, c):
                for d in range(DIM // 16):
                    sl = pl.ds(d * 16, 16)
                    rows[b][r, sl] = rows[b][r, sl] + pos_v[r, sl]
                return c

            lax.fori_loop(0, SEQ_LEN, add_row, 0, unroll=4)
            pltpu.async_copy(rows[b], out_hbm.at[base + s], ssems[b])
        return carry

    lax.fori_loop(0, spw // NB, one_group, 0)

    for b in range(NB):
        store_wait(spw - NB + b, b)


def kernel(inputs, token_table, pos_table):
    batch, seq_len = inputs.shape
    assert seq_len == SEQ_LEN and batch % (NW * NB) == 0
    idx3 = inputs.reshape(batch, 2, HALF).astype(jnp.int32)

    mesh = plsc.VectorSubcoreMesh(
        core_axis_name="c", subcore_axis_name="s",
        num_cores=NC, num_subcores=NS)

    spw = batch // NW
    run = pl.kernel(
        _body,
        out_type=jax.ShapeDtypeStruct((batch, SEQ_LEN, DIM), jnp.float32),
        mesh=mesh,
        scratch_types=[
            pltpu.VMEM((spw, 2, HALF), jnp.int32),
            pltpu.VMEM((SEQ_LEN, DIM), jnp.float32),
            [pltpu.VMEM((SEQ_LEN, DIM), jnp.float32) for _ in range(NB)],
            [pltpu.SemaphoreType.DMA for _ in range(NB)],
            [pltpu.SemaphoreType.DMA for _ in range(NB)],
        ],
        compiler_params=pltpu.CompilerParams(use_tc_tiling_on_sc=False),
    )
    return run(idx3, token_table, pos_table)
